# baseline (speedup 1.0000x reference)
import functools

import jax
import jax.numpy as jnp
from jax import lax
from jax.experimental import pallas as pl
from jax.experimental.pallas import tpu as pltpu
from jax.experimental.pallas import tpu_sc as plsc

N_NODES = 10000
N_EDGES = 320000
D = 128

NC = 2
NS = 16
NW = NC * NS
E_W = N_EDGES // NW
CHUNK = 80
NCHUNK = E_W // CHUNK
SB = 25
NSB = NCHUNK // SB
N_PAD = 640 * NS
ROWS_T = N_PAD // NS
DEG_PAD = N_PAD


def _sc_aggregate(x, row_r, col_r):
    mesh = plsc.VectorSubcoreMesh(core_axis_name="c", subcore_axis_name="s")

    @functools.partial(
        pl.kernel,
        mesh=mesh,
        out_type=(
            jax.ShapeDtypeStruct((NC, N_PAD, D), jnp.float32),
            jax.ShapeDtypeStruct((NC, DEG_PAD), jnp.float32),
        ),
        scratch_types=[
            pltpu.VMEM_SHARED((N_PAD, D), jnp.float32),
            pltpu.VMEM_SHARED((DEG_PAD,), jnp.float32),
            pltpu.VMEM((2, SB, CHUNK), jnp.int32),
            pltpu.VMEM((2, SB, CHUNK), jnp.int32),
            pltpu.VMEM((2, CHUNK, D), jnp.float32),
            pltpu.VMEM((CHUNK,), jnp.float32),
            pltpu.VMEM((640,), jnp.float32),
            pltpu.SemaphoreType.DMA((2,)),
            pltpu.SemaphoreType.DMA((2,)),
        ],
    )
    def k(x_hbm, row_hbm, col_hbm, s_out, deg_out,
          acc, deg, ridx, cidx, gbuf, ones, dz, sem, isem):
        c = lax.axis_index("c")
        s = lax.axis_index("s")
        wid = c * NS + s

        pltpu.make_async_copy(row_hbm.at[wid, 0], ridx.at[0],
                              isem.at[0]).start()
        pltpu.make_async_copy(col_hbm.at[wid, 0], cidx.at[0],
                              isem.at[0]).start()

        zero16 = jnp.zeros((16,), jnp.float32)
        one16 = jnp.ones((16,), jnp.float32)
        for i in range(CHUNK // 16):
            ones[pl.ds(i * 16, 16)] = one16

        def zg(i, _):
            gbuf[0, i // 8, pl.ds((i % 8) * 16, 16)] = zero16
            return 0
        lax.fori_loop(0, CHUNK * 8, zg, 0)

        def zd(i, _):
            dz[pl.ds(i * 16, 16)] = zero16
            return 0
        lax.fori_loop(0, 40, zd, 0)

        for j in range(ROWS_T // CHUNK):
            pltpu.sync_copy(gbuf.at[0],
                            acc.at[pl.ds(s * ROWS_T + j * CHUNK, CHUNK)])
        pltpu.sync_copy(dz, deg.at[pl.ds(s * 640, 640)])
        plsc.subcore_barrier()

        def body(j, _):
            slot = lax.rem(j, 2)
            prev = lax.rem(j + 1, 2)
            sbc = j // SB
            sbi = lax.rem(sbc, 2)
            jj = lax.rem(j, SB)
            pj = j - 1
            psbi = lax.rem(pj // SB, 2)
            pjj = lax.rem(pj, SB)

            @pl.when(jnp.logical_and(j < NCHUNK, jj != 0))
            def _():
                pltpu.make_async_copy(
                    x_hbm.at[ridx.at[sbi, jj]], gbuf.at[slot],
                    sem.at[slot]).start()

            @pl.when(j > 0)
            def _():
                pltpu.make_async_copy(
                    x_hbm.at[ridx.at[psbi, pjj]], gbuf.at[prev],
                    sem.at[prev]).wait()
                pltpu.sync_copy(gbuf.at[prev], acc.at[cidx.at[psbi, pjj]],
                                add=True)
                pltpu.sync_copy(ones, deg.at[cidx.at[psbi, pjj]], add=True)

            @pl.when(jnp.logical_and(j < NCHUNK, jj == 0))
            def _():
                pltpu.make_async_copy(row_hbm.at[wid, sbc], ridx.at[sbi],
                                      isem.at[sbi]).wait()
                pltpu.make_async_copy(col_hbm.at[wid, sbc], cidx.at[sbi],
                                      isem.at[sbi]).wait()

                @pl.when(sbc + 1 < NSB)
                def _():
                    sn = lax.rem(sbc + 1, 2)
                    pltpu.make_async_copy(row_hbm.at[wid, sbc + 1],
                                          ridx.at[sn], isem.at[sn]).start()
                    pltpu.make_async_copy(col_hbm.at[wid, sbc + 1],
                                          cidx.at[sn], isem.at[sn]).start()
                pltpu.make_async_copy(
                    x_hbm.at[ridx.at[sbi, jj]], gbuf.at[slot],
                    sem.at[slot]).start()
            return 0
        lax.fori_loop(0, NCHUNK + 1, body, 0)

        plsc.subcore_barrier()

        pltpu.sync_copy(acc.at[pl.ds(s * ROWS_T, ROWS_T)],
                        s_out.at[c, pl.ds(s * ROWS_T, ROWS_T)])
        pltpu.sync_copy(deg.at[pl.ds(s * 640, 640)],
                        deg_out.at[c, pl.ds(s * 640, 640)])

    return k(x, row_r, col_r)


BLK = 1000


def _epilogue_body(x_r, s0_r, s1_r, d0_r, d1_r, w_r, b_r, o_r):
    deg = d0_r[...] + d1_r[...]
    inv = jnp.where(deg == 0.0, 0.0, 1.0 / deg)
    h = x_r[...] - (s0_r[...] + s1_r[...]) * inv
    y = lax.dot_general(h, w_r[...], (((1,), (1,)), ((), ())),
                        preferred_element_type=jnp.float32)
    o_r[...] = jnp.maximum(y + b_r[...], 0.0)


def _tc_epilogue(x, s0, s1, d0, d1, W, b2):
    grid = (N_NODES // BLK,)
    return pl.pallas_call(
        _epilogue_body,
        grid=grid,
        in_specs=[
            pl.BlockSpec((BLK, D), lambda i: (i, 0)),
            pl.BlockSpec((BLK, D), lambda i: (i, 0)),
            pl.BlockSpec((BLK, D), lambda i: (i, 0)),
            pl.BlockSpec((BLK, 1), lambda i: (i, 0)),
            pl.BlockSpec((BLK, 1), lambda i: (i, 0)),
            pl.BlockSpec((D, D), lambda i: (0, 0)),
            pl.BlockSpec((1, D), lambda i: (0, 0)),
        ],
        out_specs=pl.BlockSpec((BLK, D), lambda i: (i, 0)),
        out_shape=jax.ShapeDtypeStruct((N_NODES, D), jnp.float32),
    )(x, s0, s1, d0, d1, W, b2)


def kernel(x, edge_index, W, b):
    ei = edge_index.astype(jnp.int32)
    row_r = ei[0].reshape(NW, NSB, SB, CHUNK)
    col_r = ei[1].reshape(NW, NSB, SB, CHUNK)
    s_part, deg_part = _sc_aggregate(x, row_r, col_r)
    s0, s1 = s_part[0, :N_NODES], s_part[1, :N_NODES]
    d0 = deg_part[0, :N_NODES].reshape(N_NODES, 1)
    d1 = deg_part[1, :N_NODES].reshape(N_NODES, 1)
    return _tc_epilogue(x, s0, s1, d0, d1, W, b.reshape(1, D))


# speedup vs baseline: 1.0478x; 1.0478x over previous
import functools

import jax
import jax.numpy as jnp
from jax import lax
from jax.experimental import pallas as pl
from jax.experimental.pallas import tpu as pltpu
from jax.experimental.pallas import tpu_sc as plsc

N_NODES = 10000
N_EDGES = 320000
D = 128

NC = 2
NS = 16
NW = NC * NS
E_W = N_EDGES // NW
CHUNK = 80
NCHUNK = E_W // CHUNK
SB = 25
NSB = NCHUNK // SB
N_PAD = 640 * NS
ROWS_T = N_PAD // NS
DEG_PAD = N_PAD


def _sc_aggregate(x, row_r, col_r):
    mesh = plsc.VectorSubcoreMesh(core_axis_name="c", subcore_axis_name="s")

    @functools.partial(
        pl.kernel,
        mesh=mesh,
        out_type=(
            jax.ShapeDtypeStruct((NC, N_PAD, D), jnp.float32),
            jax.ShapeDtypeStruct((NC, DEG_PAD), jnp.float32),
        ),
        scratch_types=[
            pltpu.VMEM_SHARED((N_PAD, D), jnp.float32),
            pltpu.VMEM_SHARED((DEG_PAD,), jnp.float32),
            pltpu.VMEM((2, SB, CHUNK), jnp.int32),
            pltpu.VMEM((2, SB, CHUNK), jnp.int32),
            pltpu.VMEM((2, CHUNK, D), jnp.float32),
            pltpu.VMEM((CHUNK,), jnp.float32),
            pltpu.VMEM((640,), jnp.float32),
            pltpu.SemaphoreType.DMA((2,)),
            pltpu.SemaphoreType.DMA((2,)),
        ],
    )
    def k(x_hbm, row_hbm, col_hbm, s_out, deg_out,
          acc, deg, ridx, cidx, gbuf, ones, dz, sem, isem):
        c = lax.axis_index("c")
        s = lax.axis_index("s")
        wid = c * NS + s

        pltpu.make_async_copy(row_hbm.at[wid, 0], ridx.at[0],
                              isem.at[0]).start()
        pltpu.make_async_copy(col_hbm.at[wid, 0], cidx.at[0],
                              isem.at[0]).start()

        zero16 = jnp.zeros((16,), jnp.float32)
        one16 = jnp.ones((16,), jnp.float32)
        for i in range(CHUNK // 16):
            ones[pl.ds(i * 16, 16)] = one16

        def zg(i, _):
            gbuf[0, i // 8, pl.ds((i % 8) * 16, 16)] = zero16
            return 0
        lax.fori_loop(0, CHUNK * 8, zg, 0)

        def zd(i, _):
            dz[pl.ds(i * 16, 16)] = zero16
            return 0
        lax.fori_loop(0, 40, zd, 0)

        for j in range(ROWS_T // CHUNK):
            pltpu.sync_copy(gbuf.at[0],
                            acc.at[pl.ds(s * ROWS_T + j * CHUNK, CHUNK)])
        pltpu.sync_copy(dz, deg.at[pl.ds(s * 640, 640)])
        plsc.subcore_barrier()

        def sb_body(sb, _):
            si = lax.rem(sb, 2)
            sn = lax.rem(sb + 1, 2)
            pltpu.make_async_copy(row_hbm.at[wid, sb], ridx.at[si],
                                  isem.at[si]).wait()
            pltpu.make_async_copy(col_hbm.at[wid, sb], cidx.at[si],
                                  isem.at[si]).wait()

            @pl.when(sb + 1 < NSB)
            def _():
                pltpu.make_async_copy(row_hbm.at[wid, sb + 1], ridx.at[sn],
                                      isem.at[sn]).start()
                pltpu.make_async_copy(col_hbm.at[wid, sb + 1], cidx.at[sn],
                                      isem.at[sn]).start()

            def body(j, _):
                slot = lax.rem(j, 2)
                prev = lax.rem(j + 1, 2)

                @pl.when(j < SB)
                def _():
                    pltpu.make_async_copy(
                        x_hbm.at[ridx.at[si, j]], gbuf.at[slot],
                        sem.at[slot]).start()

                @pl.when(j > 0)
                def _():
                    pltpu.make_async_copy(
                        x_hbm.at[ridx.at[si, j - 1]], gbuf.at[prev],
                        sem.at[prev]).wait()
                    pltpu.sync_copy(gbuf.at[prev], acc.at[cidx.at[si, j - 1]],
                                    add=True)
                    pltpu.sync_copy(ones, deg.at[cidx.at[si, j - 1]], add=True)
                return 0
            lax.fori_loop(0, SB + 1, body, 0)
            return 0
        lax.fori_loop(0, NSB, sb_body, 0)

        plsc.subcore_barrier()

        pltpu.sync_copy(acc.at[pl.ds(s * ROWS_T, ROWS_T)],
                        s_out.at[c, pl.ds(s * ROWS_T, ROWS_T)])
        pltpu.sync_copy(deg.at[pl.ds(s * 640, 640)],
                        deg_out.at[c, pl.ds(s * 640, 640)])

    return k(x, row_r, col_r)


BLK = 1000


def _epilogue_body(x_r, s0_r, s1_r, d0_r, d1_r, w_r, b_r, o_r):
    deg = d0_r[0] + d1_r[0]
    inv = jnp.where(deg == 0.0, 0.0, 1.0 / deg)
    h = x_r[...] - (s0_r[0] + s1_r[0]) * inv
    y = lax.dot_general(h, w_r[...], (((1,), (1,)), ((), ())),
                        preferred_element_type=jnp.float32)
    o_r[...] = jnp.maximum(y + b_r[...], 0.0)


def _tc_epilogue(x, s_part, deg3, W, b2):
    grid = (N_NODES // BLK,)
    return pl.pallas_call(
        _epilogue_body,
        grid=grid,
        in_specs=[
            pl.BlockSpec((BLK, D), lambda i: (i, 0)),
            pl.BlockSpec((1, BLK, D), lambda i: (0, i, 0)),
            pl.BlockSpec((1, BLK, D), lambda i: (1, i, 0)),
            pl.BlockSpec((1, BLK, 1), lambda i: (0, i, 0)),
            pl.BlockSpec((1, BLK, 1), lambda i: (1, i, 0)),
            pl.BlockSpec((D, D), lambda i: (0, 0)),
            pl.BlockSpec((1, D), lambda i: (0, 0)),
        ],
        out_specs=pl.BlockSpec((BLK, D), lambda i: (i, 0)),
        out_shape=jax.ShapeDtypeStruct((N_NODES, D), jnp.float32),
    )(x, s_part, s_part, deg3, deg3, W, b2)


def kernel(x, edge_index, W, b):
    ei = edge_index.astype(jnp.int32)
    row_r = ei[0].reshape(NW, NSB, SB, CHUNK)
    col_r = ei[1].reshape(NW, NSB, SB, CHUNK)
    s_part, deg_part = _sc_aggregate(x, row_r, col_r)
    deg3 = deg_part.reshape(NC, DEG_PAD, 1)
    return _tc_epilogue(x, s_part, deg3, W, b.reshape(1, D))


# speedup vs baseline: 1.0592x; 1.0109x over previous
import functools

import jax
import jax.numpy as jnp
from jax import lax
from jax.experimental import pallas as pl
from jax.experimental.pallas import tpu as pltpu
from jax.experimental.pallas import tpu_sc as plsc

N_NODES = 10000
N_EDGES = 320000
D = 128

NC = 2
NS = 16
NW = NC * NS
E_W = N_EDGES // NW
CHUNK = 80
NCHUNK = E_W // CHUNK
SB = 25
NSB = NCHUNK // SB
N_PAD = 640 * NS
ROWS_T = N_PAD // NS
DEG_PAD = N_PAD


def _sc_aggregate(x, row_r, col_r):
    mesh = plsc.VectorSubcoreMesh(core_axis_name="c", subcore_axis_name="s")

    @functools.partial(
        pl.kernel,
        mesh=mesh,
        out_type=(
            jax.ShapeDtypeStruct((NC, N_PAD, D), jnp.float32),
            jax.ShapeDtypeStruct((NC, DEG_PAD), jnp.float32),
        ),
        scratch_types=[
            pltpu.VMEM_SHARED((N_PAD, D), jnp.float32),
            pltpu.VMEM_SHARED((DEG_PAD,), jnp.float32),
            pltpu.VMEM((2, SB, CHUNK), jnp.int32),
            pltpu.VMEM((2, SB, CHUNK), jnp.int32),
            pltpu.VMEM((2, CHUNK, D), jnp.float32),
            pltpu.VMEM((CHUNK,), jnp.float32),
            pltpu.VMEM((640,), jnp.float32),
            pltpu.SemaphoreType.DMA((2,)),
            pltpu.SemaphoreType.DMA((2,)),
        ],
    )
    def k(x_hbm, row_hbm, col_hbm, s_out, deg_out,
          acc, deg, ridx, cidx, gbuf, ones, dz, sem, isem):
        c = lax.axis_index("c")
        s = lax.axis_index("s")
        wid = c * NS + s

        pltpu.make_async_copy(row_hbm.at[wid, 0], ridx.at[0],
                              isem.at[0]).start()
        pltpu.make_async_copy(col_hbm.at[wid, 0], cidx.at[0],
                              isem.at[0]).start()

        zero16 = jnp.zeros((16,), jnp.float32)
        one16 = jnp.ones((16,), jnp.float32)
        for i in range(CHUNK // 16):
            ones[pl.ds(i * 16, 16)] = one16

        def zg(i, _):
            gbuf[0, i // 8, pl.ds((i % 8) * 16, 16)] = zero16
            return 0
        lax.fori_loop(0, CHUNK * 8, zg, 0)

        def zd(i, _):
            dz[pl.ds(i * 16, 16)] = zero16
            return 0
        lax.fori_loop(0, 40, zd, 0)

        for j in range(ROWS_T // CHUNK):
            pltpu.sync_copy(gbuf.at[0],
                            acc.at[pl.ds(s * ROWS_T + j * CHUNK, CHUNK)])
        pltpu.sync_copy(dz, deg.at[pl.ds(s * 640, 640)])
        plsc.subcore_barrier()

        def sb_body(sb, _):
            si = lax.rem(sb, 2)
            sn = lax.rem(sb + 1, 2)
            pltpu.make_async_copy(row_hbm.at[wid, sb], ridx.at[si],
                                  isem.at[si]).wait()
            pltpu.make_async_copy(col_hbm.at[wid, sb], cidx.at[si],
                                  isem.at[si]).wait()

            @pl.when(sb + 1 < NSB)
            def _():
                pltpu.make_async_copy(row_hbm.at[wid, sb + 1], ridx.at[sn],
                                      isem.at[sn]).start()
                pltpu.make_async_copy(col_hbm.at[wid, sb + 1], cidx.at[sn],
                                      isem.at[sn]).start()

            def body(j, _):
                slot = lax.rem(j, 2)
                prev = lax.rem(j + 1, 2)

                @pl.when(j < SB)
                def _():
                    pltpu.make_async_copy(
                        x_hbm.at[ridx.at[si, j]], gbuf.at[slot],
                        sem.at[slot]).start()

                @pl.when(j > 0)
                def _():
                    pltpu.make_async_copy(
                        x_hbm.at[ridx.at[si, j - 1]], gbuf.at[prev],
                        sem.at[prev]).wait()
                    pltpu.sync_copy(gbuf.at[prev], acc.at[cidx.at[si, j - 1]],
                                    add=True)
                    pltpu.sync_copy(ones, deg.at[cidx.at[si, j - 1]], add=True)
                return 0
            lax.fori_loop(0, SB + 1, body, 0)
            return 0
        lax.fori_loop(0, NSB, sb_body, 0)

        plsc.subcore_barrier()

        pltpu.sync_copy(acc.at[pl.ds(s * ROWS_T, ROWS_T)],
                        s_out.at[c, pl.ds(s * ROWS_T, ROWS_T)])
        pltpu.sync_copy(deg.at[pl.ds(s * 640, 640)],
                        deg_out.at[c, pl.ds(s * 640, 640)])

    return k(x, row_r, col_r)


BLK = 2000


def _epilogue_body(x_r, s0_r, s1_r, d0_r, d1_r, w_r, b_r, o_r):
    deg = d0_r[0] + d1_r[0]
    inv = jnp.where(deg == 0.0, 0.0, 1.0 / deg)
    h = x_r[...] - (s0_r[0] + s1_r[0]) * inv
    y = lax.dot_general(h, w_r[...], (((1,), (1,)), ((), ())),
                        preferred_element_type=jnp.float32)
    o_r[...] = jnp.maximum(y + b_r[...], 0.0)


def _tc_epilogue(x, s_part, deg3, W, b2):
    grid = (N_NODES // BLK,)
    return pl.pallas_call(
        _epilogue_body,
        grid=grid,
        in_specs=[
            pl.BlockSpec((BLK, D), lambda i: (i, 0)),
            pl.BlockSpec((1, BLK, D), lambda i: (0, i, 0)),
            pl.BlockSpec((1, BLK, D), lambda i: (1, i, 0)),
            pl.BlockSpec((1, BLK, 1), lambda i: (0, i, 0)),
            pl.BlockSpec((1, BLK, 1), lambda i: (1, i, 0)),
            pl.BlockSpec((D, D), lambda i: (0, 0)),
            pl.BlockSpec((1, D), lambda i: (0, 0)),
        ],
        out_specs=pl.BlockSpec((BLK, D), lambda i: (i, 0)),
        out_shape=jax.ShapeDtypeStruct((N_NODES, D), jnp.float32),
    )(x, s_part, s_part, deg3, deg3, W, b2)


def kernel(x, edge_index, W, b):
    ei = edge_index.astype(jnp.int32)
    row_r = ei[0].reshape(NW, NSB, SB, CHUNK)
    col_r = ei[1].reshape(NW, NSB, SB, CHUNK)
    s_part, deg_part = _sc_aggregate(x, row_r, col_r)
    deg3 = deg_part.reshape(NC, DEG_PAD, 1)
    return _tc_epilogue(x, s_part, deg3, W, b.reshape(1, D))
